# baseline (speedup 1.0000x reference)
import functools

import jax
import jax.numpy as jnp
from jax import lax
from jax.experimental import pallas as pl
from jax.experimental.pallas import tpu as pltpu

M = 6890
N = 20000
M_PAD = 6912
N_PAD = 20480
BN = 2048
LANES = 128
PAD_VAL = 1.0e4


def _chamfer_body(scan_ref, temp_ref, out_ref):
    tsq = temp_ref[:] * temp_ref[:]
    sqt = jnp.sum(tsq, axis=1, keepdims=True)
    temp2 = (temp_ref[:] * 2.0).astype(jnp.bfloat16)
    def step(j, acc):
        blk = scan_ref[:, pl.ds(j * BN, BN)]
        s0 = blk[0:1, :]
        s1 = blk[1:2, :]
        s2 = blk[2:3, :]
        sqs = s0 * s0 + s1 * s1 + s2 * s2
        cross2 = lax.dot_general(
            temp2, blk.astype(jnp.bfloat16), (((1,), (0,)), ((), ())),
            preferred_element_type=jnp.float32)
        v = sqs - cross2
        chunks = [v[:, c * LANES:(c + 1) * LANES] for c in range(BN // LANES)]
        while len(chunks) > 1:
            chunks = [jnp.minimum(chunks[i], chunks[i + 1])
                      for i in range(0, len(chunks), 2)]
        return jnp.minimum(acc, chunks[0])

    acc0 = jnp.full((M_PAD, LANES), jnp.inf, jnp.float32)
    acc = lax.fori_loop(0, N_PAD // BN, step, acc0)

    dist2 = jnp.min(acc, axis=1, keepdims=True) + sqt
    dist2 = jnp.maximum(dist2, 0.0)
    row = lax.broadcasted_iota(jnp.int32, (M_PAD, 1), 0)
    dist2 = jnp.where(row < M, dist2, 0.0)
    out_ref[:, :] = jnp.sum(dist2, keepdims=True)


@functools.partial(jax.jit)
def kernel(scan_vertices, template_vertices):
    scan_t = jnp.pad(scan_vertices, ((0, N_PAD - N), (0, 0)),
                     constant_values=PAD_VAL).T
    temp = jnp.pad(template_vertices, ((0, M_PAD - M), (0, 0)))
    out = pl.pallas_call(
        _chamfer_body,
        out_shape=jax.ShapeDtypeStruct((1, 1), jnp.float32),
    )(scan_t, temp)
    return out[0, 0]


# speedup vs baseline: 1.0264x; 1.0264x over previous
import functools

import jax
import jax.numpy as jnp
from jax import lax
from jax.experimental import pallas as pl
from jax.experimental.pallas import tpu as pltpu

M = 6890
N = 20000
M_PAD = 6912
N_PAD = 20096
BN = 2048
BN_TAIL = N_PAD - (N_PAD // BN) * BN
LANES = 128
PAD_VAL = 1.0e4


def _chamfer_body(scan_ref, temp_ref, out_ref):
    tsq = temp_ref[:] * temp_ref[:]
    sqt = jnp.sum(tsq, axis=1, keepdims=True)
    temp2 = temp_ref[:] * 2.0
    def block_min(base, bn, acc):
        blk = scan_ref[:, pl.ds(base, bn)]
        s0 = blk[0:1, :]
        s1 = blk[1:2, :]
        s2 = blk[2:3, :]
        sqs = s0 * s0 + s1 * s1 + s2 * s2
        cross2 = lax.dot_general(
            temp2, blk, (((1,), (0,)), ((), ())),
            preferred_element_type=jnp.float32)
        v = sqs - cross2
        chunks = [v[:, c * LANES:(c + 1) * LANES] for c in range(bn // LANES)]
        while len(chunks) > 1:
            nxt = [jnp.minimum(chunks[i], chunks[i + 1])
                   for i in range(0, len(chunks) - 1, 2)]
            if len(chunks) % 2:
                nxt.append(chunks[-1])
            chunks = nxt
        return jnp.minimum(acc, chunks[0])

    acc0 = jnp.full((M_PAD, LANES), jnp.inf, jnp.float32)
    acc = lax.fori_loop(0, N_PAD // BN,
                        lambda j, a: block_min(j * BN, BN, a), acc0)
    if BN_TAIL:
        acc = block_min((N_PAD // BN) * BN, BN_TAIL, acc)

    dist2 = jnp.min(acc, axis=1, keepdims=True) + sqt
    dist2 = jnp.maximum(dist2, 0.0)
    row = lax.broadcasted_iota(jnp.int32, (M_PAD, 1), 0)
    dist2 = jnp.where(row < M, dist2, 0.0)
    out_ref[:, :] = jnp.sum(dist2, keepdims=True)


@functools.partial(jax.jit)
def kernel(scan_vertices, template_vertices):
    scan_t = jnp.pad(scan_vertices, ((0, N_PAD - N), (0, 0)),
                     constant_values=PAD_VAL).T
    temp = jnp.pad(template_vertices, ((0, M_PAD - M), (0, 0)))
    out = pl.pallas_call(
        _chamfer_body,
        out_shape=jax.ShapeDtypeStruct((1, 1), jnp.float32),
    )(scan_t, temp)
    return out[0, 0]
